# Initial kernel scaffold; baseline (speedup 1.0000x reference)
import jax
import jax.numpy as jnp
from jax.experimental import pallas as pl


def kernel(x, sW1, sb1, sW2, sb2, eW1, eb1, eW2, eb2, gW, gb, gate_bias):
    raise NotImplementedError("write your pallas kernel here")


# speedup vs baseline: 1.2929x; 1.2929x over previous
import jax
import jax.numpy as jnp
from jax.experimental import pallas as pl
from jax.experimental.pallas import tpu as pltpu

N_TOK = 2048
C = 768
H = 3072
E = 8
T_V1 = 512


def _mlp(xr, W1, b1, W2, b2):
    xb = xr.astype(jnp.bfloat16)
    h = jax.lax.dot_general(
        xb, W1.astype(jnp.bfloat16),
        (((1,), (1,)), ((), ())), preferred_element_type=jnp.float32)
    h = jnp.maximum(h + b1, 0.0)
    y = jax.lax.dot_general(
        h.astype(jnp.bfloat16), W2.astype(jnp.bfloat16),
        (((1,), (1,)), ((), ())), preferred_element_type=jnp.float32)
    return y + b2


def _router_eid(xr, gW, gv):
    logits = jax.lax.dot_general(
        xr, gW, (((1,), (1,)), ((), ())),
        preferred_element_type=jnp.float32) + gv
    m = jnp.max(logits, axis=-1, keepdims=True)
    iota = jax.lax.broadcasted_iota(jnp.int32, logits.shape, 1)
    return jnp.min(jnp.where(logits == m, iota, E), axis=-1)


def _shared_body(x_ref, W1_ref, b1_ref, W2_ref, b2_ref, out_ref):
    out_ref[...] = _mlp(x_ref[...], W1_ref[...], b1_ref[0],
                        W2_ref[...], b2_ref[0])


def _routed_dense_body(x_ref, W1c_ref, b1c_ref, W2c_ref, b2_ref,
                       gW_ref, gv_ref, shared_ref, out_ref, eid_scr):
    e = pl.program_id(0)
    hh = pl.program_id(1)

    @pl.when((e == 0) & (hh == 0))
    def _():
        eid = _router_eid(x_ref[...], gW_ref[...], gv_ref[...])
        eid_scr[...] = eid[:, None]
        out_ref[...] = shared_ref[...]

    w = (eid_scr[...] == e).astype(jnp.float32)
    xb = x_ref[...].astype(jnp.bfloat16)
    h = jax.lax.dot_general(
        xb, W1c_ref[0, 0].astype(jnp.bfloat16),
        (((1,), (1,)), ((), ())), preferred_element_type=jnp.float32)
    h = jnp.maximum(h + b1c_ref[0, 0], 0.0)
    part = jax.lax.dot_general(
        h.astype(jnp.bfloat16), W2c_ref[0].astype(jnp.bfloat16),
        (((1,), (1,)), ((), ())), preferred_element_type=jnp.float32)
    first = (hh == 0).astype(jnp.float32)
    part = part + b2_ref[0, 0] * first
    out_ref[...] += part * w


def kernel(x, sW1, sb1, sW2, sb2, eW1, eb1, eW2, eb2, gW, gb, gate_bias):
    xs = x.reshape(N_TOK, C)
    gv = (gb + gate_bias).reshape(1, E)
    nt = N_TOK // T_V1

    shared_out = pl.pallas_call(
        _shared_body,
        grid=(nt,),
        in_specs=[
            pl.BlockSpec((T_V1, C), lambda t: (t, 0)),
            pl.BlockSpec((H, C), lambda t: (0, 0)),
            pl.BlockSpec((1, H), lambda t: (0, 0)),
            pl.BlockSpec((C, H), lambda t: (0, 0)),
            pl.BlockSpec((1, C), lambda t: (0, 0)),
        ],
        out_specs=pl.BlockSpec((T_V1, C), lambda t: (t, 0)),
        out_shape=jax.ShapeDtypeStruct((N_TOK, C), jnp.float32),
    )(xs, sW1, sb1.reshape(1, H), sW2, sb2.reshape(1, C))

    HC = 768
    nh = H // HC
    out = pl.pallas_call(
        _routed_dense_body,
        grid=(E, nh),
        in_specs=[
            pl.BlockSpec((N_TOK, C), lambda e, h: (0, 0)),
            pl.BlockSpec((1, 1, HC, C), lambda e, h: (e, h, 0, 0)),
            pl.BlockSpec((1, 1, HC), lambda e, h: (e, 0, h)),
            pl.BlockSpec((1, C, HC), lambda e, h: (e, 0, h)),
            pl.BlockSpec((1, 1, C), lambda e, h: (e, 0, 0)),
            pl.BlockSpec((E, C), lambda e, h: (0, 0)),
            pl.BlockSpec((1, E), lambda e, h: (0, 0)),
            pl.BlockSpec((N_TOK, C), lambda e, h: (0, 0)),
        ],
        out_specs=pl.BlockSpec((N_TOK, C), lambda e, h: (0, 0)),
        out_shape=jax.ShapeDtypeStruct((N_TOK, C), jnp.float32),
        scratch_shapes=[pltpu.VMEM((N_TOK, 1), jnp.int32)],
    )(xs, eW1.reshape(E, nh, HC, C), eb1.reshape(E, 1, H),
      eW2, eb2.reshape(E, 1, C), gW, gv, shared_out)

    return out.reshape(x.shape)
